# Initial kernel scaffold; baseline (speedup 1.0000x reference)
import jax
import jax.numpy as jnp
from jax.experimental import pallas as pl


def kernel(input):
    raise NotImplementedError("write your pallas kernel here")


# speedup vs baseline: 38.7723x; 38.7723x over previous
import jax
import jax.numpy as jnp
from jax.experimental import pallas as pl

_ROWS = 128
_COLS = 32768
_BLOCK_ROWS = 16
_NITER = 12


def _sparsemax_block(x_ref, o_ref):
    x = x_ref[...]
    m = jnp.max(x, axis=1, keepdims=True)
    y = x - m
    t = jnp.full_like(m, -1.0)
    for _ in range(_NITER):
        gt = y > t
        s = jnp.sum(jnp.where(gt, y, 0.0), axis=1, keepdims=True)
        n = jnp.sum(gt.astype(jnp.float32), axis=1, keepdims=True)
        t = (s - 1.0) / n
    o_ref[...] = jnp.maximum(y - t, 0.0)


def kernel(input):
    return pl.pallas_call(
        _sparsemax_block,
        grid=(_ROWS // _BLOCK_ROWS,),
        in_specs=[pl.BlockSpec((_BLOCK_ROWS, _COLS), lambda i: (i, 0))],
        out_specs=pl.BlockSpec((_BLOCK_ROWS, _COLS), lambda i: (i, 0)),
        out_shape=jax.ShapeDtypeStruct((_ROWS, _COLS), jnp.float32),
    )(input)
